# baseline (speedup 1.0000x reference)
import functools

import jax
import jax.numpy as jnp
from jax.experimental import pallas as pl
from jax.experimental.pallas import tpu as pltpu


def _sep_kernel(wdw_ref, wpw_ref, x_ref, o_ref, xq_ref, *, k, pad, h, w, t):
    m, cin = x_ref.shape[1], x_ref.shape[2]
    cout = o_ref.shape[2]
    rt = h // t
    wp = w + 2 * pad

    @pl.when(pl.program_id(1) == 0)
    def _build_padded():
        xb = x_ref[...].astype(jnp.bfloat16).reshape(h, w, m, cin)
        zr = jnp.zeros((pad, wp, m, cin), jnp.bfloat16)
        zc = jnp.zeros((h, pad, m, cin), jnp.bfloat16)
        xq_ref[0:pad] = zr
        xq_ref[h + pad:h + 2 * pad] = zr
        xq_ref[pad:h + pad, 0:pad] = zc
        xq_ref[pad:h + pad, w + pad:wp] = zc
        xq_ref[pad:h + pad, pad:w + pad] = xb

    r0 = pl.program_id(1) * rt
    acc = None
    for kh in range(k):
        rows = xq_ref[pl.ds(r0 + kh, rt)]
        for kw in range(k):
            tap = rows[:, kw:kw + w]
            term = tap * wdw_ref[kh * k + kw]
            acc = term if acc is None else acc + term

    dw2 = acc.reshape(rt * w * m, cin)
    out = jnp.dot(dw2, wpw_ref[...], preferred_element_type=jnp.float32)
    o_ref[...] = out.reshape(rt * w, m, cout)


def kernel(x_nchw, w_dw, w_pw):
    n, cin, h, w = x_nchw.shape
    k = w_dw.shape[2]
    pad = (k - 1) // 2
    cout = w_pw.shape[0]
    hw = h * w

    m = 16
    while n % m:
        m //= 2
    t = 2
    while h % t:
        t -= 1

    wdw_v = jnp.transpose(w_dw.reshape(cin, k * k), (1, 0)).reshape(k * k, 1, cin)
    wdw_v = wdw_v.astype(jnp.bfloat16)
    wpw_v = jnp.transpose(w_pw.reshape(cout, cin), (1, 0)).astype(jnp.bfloat16)

    xt = jnp.transpose(x_nchw, (2, 3, 0, 1)).reshape(hw, n, cin)

    body = functools.partial(_sep_kernel, k=k, pad=pad, h=h, w=w, t=t)

    out3 = pl.pallas_call(
        body,
        out_shape=jax.ShapeDtypeStruct((hw, n, cout), x_nchw.dtype),
        grid=(n // m, t),
        in_specs=[
            pl.BlockSpec((k * k, 1, cin), lambda b, j: (0, 0, 0)),
            pl.BlockSpec((cin, cout), lambda b, j: (0, 0)),
            pl.BlockSpec((hw, m, cin), lambda b, j: (0, b, 0)),
        ],
        out_specs=pl.BlockSpec((hw // t, m, cout), lambda b, j: (j, b, 0)),
        scratch_shapes=[
            pltpu.VMEM((h + 2 * pad, w + 2 * pad, m, cin), jnp.bfloat16)],
        compiler_params=pltpu.CompilerParams(
            dimension_semantics=("parallel", "arbitrary"),
            vmem_limit_bytes=56 * 2 ** 20),
    )(wdw_v, wpw_v, xt)

    return jnp.transpose(out3.reshape(h, w, n, cout), (2, 3, 0, 1))


# speedup vs baseline: 1.3831x; 1.3831x over previous
import functools

import jax
import jax.numpy as jnp
from jax.experimental import pallas as pl
from jax.experimental.pallas import tpu as pltpu


def _sep_kernel(wdw_ref, wpw_ref, x_ref, o_ref, *, k, pad, h, w):
    m, cin = x_ref.shape[1], x_ref.shape[2]
    cout = o_ref.shape[2]

    xb = x_ref[...].astype(jnp.bfloat16).reshape(h, w, m, cin)
    zc = jnp.zeros((h, pad, m, cin), jnp.bfloat16)
    zr = jnp.zeros((pad, w + 2 * pad, m, cin), jnp.bfloat16)
    xq = jnp.concatenate([zc, xb, zc], axis=1)
    xq = jnp.concatenate([zr, xq, zr], axis=0)

    acc = None
    for kh in range(k):
        for kw in range(k):
            tap = xq[kh:kh + h, kw:kw + w]
            term = tap * wdw_ref[kh * k + kw]
            acc = term if acc is None else acc + term

    dw2 = acc.reshape(h * w * m, cin)
    out = jnp.dot(dw2, wpw_ref[...], preferred_element_type=jnp.float32)
    o_ref[...] = out.reshape(h * w, m, cout)


def kernel(x_nchw, w_dw, w_pw):
    n, cin, h, w = x_nchw.shape
    k = w_dw.shape[2]
    pad = (k - 1) // 2
    cout = w_pw.shape[0]
    hw = h * w

    m = 16
    while n % m:
        m //= 2

    wdw_v = jnp.transpose(w_dw.reshape(cin, k * k), (1, 0)).reshape(k * k, 1, cin)
    wdw_v = wdw_v.astype(jnp.bfloat16)
    wpw_v = jnp.transpose(w_pw.reshape(cout, cin), (1, 0)).astype(jnp.bfloat16)

    xt = jnp.transpose(x_nchw, (2, 3, 0, 1)).reshape(hw, n, cin)

    body = functools.partial(_sep_kernel, k=k, pad=pad, h=h, w=w)

    out3 = pl.pallas_call(
        body,
        out_shape=jax.ShapeDtypeStruct((hw, n, cout), x_nchw.dtype),
        grid=(n // m,),
        in_specs=[
            pl.BlockSpec((k * k, 1, cin), lambda b: (0, 0, 0)),
            pl.BlockSpec((cin, cout), lambda b: (0, 0)),
            pl.BlockSpec((hw, m, cin), lambda b: (0, b, 0)),
        ],
        out_specs=pl.BlockSpec((hw, m, cout), lambda b: (0, b, 0)),
        compiler_params=pltpu.CompilerParams(
            dimension_semantics=("parallel",),
            vmem_limit_bytes=56 * 2 ** 20),
    )(wdw_v, wpw_v, xt)

    return jnp.transpose(out3.reshape(h, w, n, cout), (2, 3, 0, 1))


# speedup vs baseline: 1.5451x; 1.1172x over previous
import functools

import jax
import jax.numpy as jnp
from jax.experimental import pallas as pl
from jax.experimental.pallas import tpu as pltpu


def _sep_kernel(wdw_ref, wpw_ref, x_ref, o_ref, *, k, pad, h, w):
    m, cin = x_ref.shape[1], x_ref.shape[2]
    cout = o_ref.shape[2]

    wdw = wdw_ref[...].astype(jnp.bfloat16)
    wpw = wpw_ref[...].astype(jnp.bfloat16)

    xb = x_ref[...].astype(jnp.bfloat16).reshape(h, w, m, cin)
    zc = jnp.zeros((h, pad, m, cin), jnp.bfloat16)
    zr = jnp.zeros((pad, w + 2 * pad, m, cin), jnp.bfloat16)
    xq = jnp.concatenate([zc, xb, zc], axis=1)
    xq = jnp.concatenate([zr, xq, zr], axis=0)

    acc = None
    for kh in range(k):
        for kw in range(k):
            tap = xq[kh:kh + h, kw:kw + w]
            term = tap * wdw[kh * k + kw]
            acc = term if acc is None else acc + term

    dw2 = acc.reshape(h * w * m, cin)
    out = jax.lax.dot_general(dw2, wpw, (((1,), (1,)), ((), ())),
                              preferred_element_type=jnp.float32)
    o_ref[...] = out.reshape(h * w, m, cout)


def kernel(x_nchw, w_dw, w_pw):
    n, cin, h, w = x_nchw.shape
    k = w_dw.shape[2]
    pad = (k - 1) // 2
    cout = w_pw.shape[0]
    hw = h * w

    m = 16
    while n % m:
        m //= 2

    wdw_v = jnp.transpose(w_dw, (2, 3, 1, 0)).reshape(k * k, 1, cin)
    wpw_v = w_pw.reshape(cout, cin)

    xt = jnp.transpose(x_nchw, (2, 3, 0, 1)).reshape(hw, n, cin)

    body = functools.partial(_sep_kernel, k=k, pad=pad, h=h, w=w)

    out3 = pl.pallas_call(
        body,
        out_shape=jax.ShapeDtypeStruct((hw, n, cout), x_nchw.dtype),
        grid=(n // m,),
        in_specs=[
            pl.BlockSpec((k * k, 1, cin), lambda b: (0, 0, 0)),
            pl.BlockSpec((cout, cin), lambda b: (0, 0)),
            pl.BlockSpec((hw, m, cin), lambda b: (0, b, 0)),
        ],
        out_specs=pl.BlockSpec((hw, m, cout), lambda b: (0, b, 0)),
        compiler_params=pltpu.CompilerParams(
            dimension_semantics=("parallel",),
            vmem_limit_bytes=56 * 2 ** 20),
    )(wdw_v, wpw_v, xt)

    return jnp.transpose(out3.reshape(h, w, n, cout), (2, 3, 0, 1))
